# Initial kernel scaffold; baseline (speedup 1.0000x reference)
import jax
import jax.numpy as jnp
from jax.experimental import pallas as pl


def kernel(label, proto):
    raise NotImplementedError("write your pallas kernel here")


# speedup vs baseline: 1.3965x; 1.3965x over previous
import functools

import jax
import jax.numpy as jnp
from jax import lax
from jax.experimental import pallas as pl
from jax.experimental.pallas import tpu as pltpu
from jax.experimental.pallas import tpu_sc as plsc

FEAT = 1024
NCLS = 1000
BATCH = 16384


def _transpose_body(p_ref, t_ref):
    t_ref[...] = p_ref[...].T


def _transpose(proto):
    return pl.pallas_call(
        _transpose_body,
        out_shape=jax.ShapeDtypeStruct((NCLS, FEAT), jnp.float32),
    )(proto)


_info = plsc.get_sparse_core_info()
_NC = _info.num_cores
_NS = _info.num_subcores
_NW = _NC * _NS
_BPW = BATCH // _NW
_CH = 64
_NCHUNK = _BPW // _CH

_mesh = plsc.VectorSubcoreMesh(core_axis_name="c", subcore_axis_name="s")


@functools.partial(
    pl.kernel,
    mesh=_mesh,
    out_type=jax.ShapeDtypeStruct((BATCH, FEAT), jnp.float32),
    scratch_types=[
        pltpu.VMEM((_BPW,), jnp.int32),
        pltpu.VMEM((_CH, FEAT), jnp.float32),
        pltpu.SemaphoreType.DMA,
    ],
)
def _gather(table_hbm, idx_hbm, out_hbm, idx_v, buf, gsem):
    wid = lax.axis_index("s") * _NC + lax.axis_index("c")
    base = wid * _BPW
    pltpu.sync_copy(idx_hbm.at[pl.ds(base, _BPW)], idx_v)
    for i in range(_NCHUNK):
        pltpu.async_copy(
            table_hbm.at[idx_v.at[pl.ds(i * _CH, _CH)]], buf, gsem
        ).wait()
        pltpu.sync_copy(buf, out_hbm.at[pl.ds(base + i * _CH, _CH)])


def kernel(label, proto):
    table = _transpose(proto)
    return _gather(table, label)


# speedup vs baseline: 1.4806x; 1.0602x over previous
import functools

import jax
import jax.numpy as jnp
from jax import lax
from jax.experimental import pallas as pl
from jax.experimental.pallas import tpu as pltpu
from jax.experimental.pallas import tpu_sc as plsc

FEAT = 1024
NCLS = 1000
BATCH = 16384


def _transpose_body(p_ref, t_ref):
    t_ref[...] = p_ref[...].T


def _transpose(proto):
    return pl.pallas_call(
        _transpose_body,
        out_shape=jax.ShapeDtypeStruct((NCLS, FEAT), jnp.float32),
    )(proto)


_info = plsc.get_sparse_core_info()
_NC = _info.num_cores
_NS = _info.num_subcores
_NW = _NC * _NS
_BPW = BATCH // _NW
_CH = 32
_NCHUNK = _BPW // _CH

_mesh = plsc.VectorSubcoreMesh(core_axis_name="c", subcore_axis_name="s")


@functools.partial(
    pl.kernel,
    mesh=_mesh,
    out_type=jax.ShapeDtypeStruct((BATCH, FEAT), jnp.float32),
    scratch_types=[
        pltpu.VMEM((_BPW,), jnp.int32),
        pltpu.VMEM((_CH, FEAT), jnp.float32),
        pltpu.VMEM((_CH, FEAT), jnp.float32),
        pltpu.SemaphoreType.DMA,
        pltpu.SemaphoreType.DMA,
    ],
)
def _gather(table_hbm, idx_hbm, out_hbm, idx_v, buf0, buf1, gsem, ssem):
    wid = lax.axis_index("s") * _NC + lax.axis_index("c")
    base = wid * _BPW
    pltpu.sync_copy(idx_hbm.at[pl.ds(base, _BPW)], idx_v)
    bufs = (buf0, buf1)
    gathers = [None] * _NCHUNK
    stores = [None] * _NCHUNK
    gathers[0] = pltpu.async_copy(
        table_hbm.at[idx_v.at[pl.ds(0, _CH)]], bufs[0], gsem
    )
    for i in range(_NCHUNK):
        if i >= 1:
            stores[i - 1].wait()
        if i + 1 < _NCHUNK:
            gathers[i + 1] = pltpu.async_copy(
                table_hbm.at[idx_v.at[pl.ds((i + 1) * _CH, _CH)]],
                bufs[(i + 1) % 2],
                gsem,
            )
        gathers[i].wait()
        stores[i] = pltpu.async_copy(
            bufs[i % 2], out_hbm.at[pl.ds(base + i * _CH, _CH)], ssem
        )
    stores[_NCHUNK - 1].wait()


def kernel(label, proto):
    table = _transpose(proto)
    return _gather(table, label)


# speedup vs baseline: 1.6028x; 1.0826x over previous
import functools

import jax
import jax.numpy as jnp
from jax import lax
from jax.experimental import pallas as pl
from jax.experimental.pallas import tpu as pltpu
from jax.experimental.pallas import tpu_sc as plsc

FEAT = 1024
NCLS = 1000
BATCH = 16384


def _transpose_body(p_ref, t_ref):
    t_ref[...] = p_ref[...].T


def _transpose(proto):
    return pl.pallas_call(
        _transpose_body,
        out_shape=jax.ShapeDtypeStruct((NCLS, FEAT), jnp.float32),
    )(proto)


_info = plsc.get_sparse_core_info()
_NC = _info.num_cores
_NS = _info.num_subcores
_NW = _NC * _NS
_BPW = BATCH // _NW
_CH = 32
_NCHUNK = _BPW // _CH

_mesh = plsc.VectorSubcoreMesh(core_axis_name="c", subcore_axis_name="s")


@functools.partial(
    pl.kernel,
    mesh=_mesh,
    out_type=jax.ShapeDtypeStruct((BATCH, FEAT), jnp.float32),
    scratch_types=[
        pltpu.VMEM((_BPW,), jnp.int32),
        pltpu.VMEM((_CH, FEAT), jnp.float32),
        pltpu.VMEM((_CH, FEAT), jnp.float32),
        pltpu.SemaphoreType.DMA,
        pltpu.SemaphoreType.DMA,
    ],
)
def _gather(table_hbm, idx_hbm, out_hbm, idx_v, buf0, buf1, gsem, ssem):
    wid = lax.axis_index("s") * _NC + lax.axis_index("c")
    base = wid * _BPW
    pltpu.sync_copy(idx_hbm.at[pl.ds(base, _BPW)], idx_v)
    bufs = (buf0, buf1)
    gathers = [None] * _NCHUNK
    stores = [None] * _NCHUNK
    gathers[0] = pltpu.async_copy(
        table_hbm.at[idx_v.at[pl.ds(0, _CH)]], bufs[0], gsem
    )
    for i in range(_NCHUNK):
        if i >= 1:
            stores[i - 1].wait()
        if i + 1 < _NCHUNK:
            gathers[i + 1] = pltpu.async_copy(
                table_hbm.at[idx_v.at[pl.ds((i + 1) * _CH, _CH)]],
                bufs[(i + 1) % 2],
                gsem,
            )
        gathers[i].wait()
        stores[i] = pltpu.async_copy(
            bufs[i % 2], out_hbm.at[pl.ds(base + i * _CH, _CH)], ssem
        )
    stores[_NCHUNK - 1].wait()


def kernel(label, proto):
    table = proto.T
    return _gather(table, label)
